# baseline (speedup 1.0000x reference)
import functools

import jax
import jax.numpy as jnp
from jax import lax
from jax.experimental import pallas as pl
from jax.experimental.pallas import tpu as pltpu
from jax.experimental.pallas import tpu_sc as plsc

_NC = 2
_NS = 16
_NW = _NC * _NS
_L = 16


def _k1_restructure(v, tblt_hbm, tail_hbm, tblp_hbm, tins, touts, sems):
    wid = lax.axis_index("s") * _NC + lax.axis_index("c")
    nu = v // 128
    t_max = nu // _NW + 2

    iotav = lax.iota(jnp.int32, _L)

    def fire(t, b):
        u = wid + _NW * t

        @pl.when(u < nu)
        def _():
            off = pl.multiple_of(128 * u, 128)
            pltpu.async_copy(tblt_hbm.at[:, pl.ds(off, 128)],
                             tins[b].at[:, pl.ds(0, 128)], sems[b])

    def compute(b, npp):
        def pp_body(pp, _):
            c0 = jnp.full((_L,), 2 * pp, jnp.int32)
            c1 = c0 + 1
            for k in range(4):
                rvec = iotav + (16 * k)
                touts[b][pp, pl.ds(16 * k, _L)] = plsc.load_gather(
                    tins[b], [rvec, c0])
                touts[b][pp, pl.ds(64 + 16 * k, _L)] = plsc.load_gather(
                    tins[b], [rvec, c1])
            return 0

        lax.fori_loop(0, npp, pp_body, 0, unroll=8)

    for b in range(2):
        fire(b, b)

    def outer(kk, _):
        for b in range(2):
            t = 2 * kk + b
            u = wid + _NW * t

            @pl.when(u < nu)
            def _():
                pltpu.make_async_copy(
                    tblt_hbm.at[:, pl.ds(pl.multiple_of(128 * u, 128), 128)],
                    tins[b].at[:, pl.ds(0, 128)], sems[b]).wait()
                compute(b, 64)
                pltpu.sync_copy(
                    touts[b],
                    tblp_hbm.at[pl.ds(pl.multiple_of(64 * u, 8), 64)])

            fire(t + 2, b)
        return 0

    lax.fori_loop(0, t_max // 2, outer, 0)

    ntail = v - 128 * nu

    @pl.when(wid == 4)
    def _():
        pltpu.sync_copy(tail_hbm, tins[0].at[:, pl.ds(0, 128)])
        compute(0, ntail // 2)
        pltpu.sync_copy(
            touts[0].at[pl.ds(0, ntail // 2)],
            tblp_hbm.at[pl.ds(64 * nu, ntail // 2)])


def _k2_gather(batch, seq, vocab, ids_hbm, tblp_hbm, outt_hbm,
               idb, pidxs, hbufs, pairs, touts, sems):
    wid = lax.axis_index("s") * _NC + lax.axis_index("c")
    bpw = batch // _NW
    n_units = 4 * seq

    iotav = lax.iota(jnp.int32, _L)
    iota_s = iotav * seq

    pltpu.sync_copy(
        ids_hbm.at[pl.ds(pl.multiple_of(bpw * seq * wid, 1024), bpw * seq)],
        idb)

    def stage_fire(t, slot):
        bb = t // seq
        s = t - bb * seq
        for g in range(8):
            offs = iota_s + ((bb * 128 + 16 * g) * seq + s)
            idvec = plsc.load_gather(idb, [offs])
            idvec = jnp.where(idvec >= vocab, 0, idvec)
            pidxs[slot][pl.ds(16 * g, _L)] = idvec >> 1
            hbufs[slot][pl.ds(16 * g, _L)] = (idvec & 1) * 64
        pltpu.async_copy(tblp_hbm.at[pidxs[slot]], pairs[slot], sems[slot])

    def drain(t, slot):
        bb = t // seq
        s = t - bb * seq
        pltpu.make_async_copy(tblp_hbm.at[pidxs[slot]], pairs[slot],
                              sems[slot]).wait()
        def g_body(g, _):
            hvec = hbufs[slot][pl.ds(g * _L, _L)]
            for tt in range(16):
                r = g * _L + tt
                h = hvec[tt]
                cvec = jnp.full((_L,), tt, jnp.int32) + (g * _L)
                for k in range(4):
                    vals = pairs[slot][r, pl.ds(h + 16 * k, _L)]
                    plsc.store_scatter(touts[slot],
                                       [iotav + (16 * k), cvec], vals)
            return 0

        lax.fori_loop(0, 8, g_body, 0)
        boff = pl.multiple_of((4 * wid + bb) * 128, 128)
        pltpu.sync_copy(touts[slot].at[:, pl.ds(0, 128)],
                        outt_hbm.at[s, :, pl.ds(boff, 128)])

    for b in range(2):
        stage_fire(b, b)

    def outer(kk, _):
        for b in range(2):
            t = 2 * kk + b
            drain(t, b)
            nt = t + 2

            @pl.when(nt < n_units)
            def _():
                stage_fire(nt, b)
        return 0

    lax.fori_loop(0, n_units // 2, outer, 0)


def kernel(input_ids, table):
    b, s = input_ids.shape
    v, d = table.shape
    n = b * s
    assert d == 64 and v % 2 == 0 and b % (128 * _NW) == 0

    tblt = table.T
    vt = (v // 128) * 128
    tail = jnp.pad(table[vt:].T, ((0, 0), (0, 128 - (v - vt))))
    ids = input_ids.reshape(n)

    mesh = plsc.VectorSubcoreMesh(core_axis_name="c", subcore_axis_name="s",
                                  num_cores=_NC, num_subcores=_NS)
    params = pltpu.CompilerParams(use_tc_tiling_on_sc=True, needs_layout_passes=False)

    k1 = pl.kernel(
        functools.partial(_k1_restructure, v),
        out_type=jax.ShapeDtypeStruct((v // 2, 128), jnp.float32),
        mesh=mesh,
        scratch_types=[
            [pltpu.VMEM((d, 137), jnp.float32)] * 2,
            [pltpu.VMEM((64, 128), jnp.float32)] * 2,
            [pltpu.SemaphoreType.DMA] * 2,
        ],
        compiler_params=params,
    )
    tblp = k1(tblt, tail)

    k2 = pl.kernel(
        functools.partial(_k2_gather, b, s, v),
        out_type=jax.ShapeDtypeStruct((s, d, b), jnp.float32),
        mesh=mesh,
        scratch_types=[
            pltpu.VMEM((n // _NW,), jnp.int32),
            [pltpu.VMEM((128,), jnp.int32)] * 2,
            [pltpu.VMEM((128,), jnp.int32)] * 2,
            [pltpu.VMEM((128, 128), jnp.float32)] * 2,
            [pltpu.VMEM((d, 137), jnp.float32)] * 2,
            [pltpu.SemaphoreType.DMA] * 2,
        ],
        compiler_params=params,
    )
    outt = k2(ids, tblp)
    return outt.transpose(2, 0, 1)


# speedup vs baseline: 1.5512x; 1.5512x over previous
import functools

import jax
import jax.numpy as jnp
from jax import lax
from jax.experimental import pallas as pl
from jax.experimental.pallas import tpu as pltpu
from jax.experimental.pallas import tpu_sc as plsc

_NC = 2
_NS = 16
_NW = _NC * _NS
_L = 16


def _k1_restructure(v, tblt_hbm, tail_hbm, tblp_hbm, tins, touts, sems):
    wid = lax.axis_index("s") * _NC + lax.axis_index("c")
    nu = v // 128
    t_max = nu // _NW + 2

    iotav = lax.iota(jnp.int32, _L)

    def fire(t, b):
        u = wid + _NW * t

        @pl.when(u < nu)
        def _():
            off = pl.multiple_of(128 * u, 128)
            pltpu.async_copy(tblt_hbm.at[:, pl.ds(off, 128)],
                             tins[b].at[:, pl.ds(0, 128)], sems[b])

    def compute(b, npp):
        @plsc.parallel_loop(0, npp, unroll=4)
        def _(pp):
            c0 = jnp.full((_L,), 2 * pp, jnp.int32)
            c1 = c0 + 1
            for k in range(4):
                rvec = iotav + (16 * k)
                touts[b][pp, pl.ds(16 * k, _L)] = plsc.load_gather(
                    tins[b], [rvec, c0])
                touts[b][pp, pl.ds(64 + 16 * k, _L)] = plsc.load_gather(
                    tins[b], [rvec, c1])

    for b in range(2):
        fire(b, b)

    def outer(kk, _):
        for b in range(2):
            t = 2 * kk + b
            u = wid + _NW * t

            @pl.when(u < nu)
            def _():
                pltpu.make_async_copy(
                    tblt_hbm.at[:, pl.ds(pl.multiple_of(128 * u, 128), 128)],
                    tins[b].at[:, pl.ds(0, 128)], sems[b]).wait()
                compute(b, 64)
                pltpu.sync_copy(
                    touts[b],
                    tblp_hbm.at[pl.ds(pl.multiple_of(64 * u, 8), 64)])

            fire(t + 2, b)
        return 0

    lax.fori_loop(0, t_max // 2, outer, 0)

    ntail = v - 128 * nu

    @pl.when(wid == 4)
    def _():
        pltpu.sync_copy(tail_hbm, tins[0].at[:, pl.ds(0, 128)])
        compute(0, ntail // 2)
        pltpu.sync_copy(
            touts[0].at[pl.ds(0, ntail // 2)],
            tblp_hbm.at[pl.ds(64 * nu, ntail // 2)])


def _k2_gather(batch, seq, vocab, ids_hbm, tblp_hbm, outt_hbm,
               idb, pidxs, hbufs, pairs, touts, sems):
    wid = lax.axis_index("s") * _NC + lax.axis_index("c")
    bpw = batch // _NW
    n_units = 4 * seq

    iotav = lax.iota(jnp.int32, _L)
    iota_s = iotav * seq

    pltpu.sync_copy(
        ids_hbm.at[pl.ds(pl.multiple_of(bpw * seq * wid, 1024), bpw * seq)],
        idb)

    def stage_fire(t, slot):
        bb = t // seq
        s = t - bb * seq
        for g in range(8):
            offs = iota_s + ((bb * 128 + 16 * g) * seq + s)
            idvec = plsc.load_gather(idb, [offs])
            idvec = jnp.where(idvec >= vocab, 0, idvec)
            pidxs[slot][pl.ds(16 * g, _L)] = idvec >> 1
            hbufs[slot][pl.ds(16 * g, _L)] = (idvec & 1) * 64
        pltpu.async_copy(tblp_hbm.at[pidxs[slot]], pairs[slot], sems[slot])

    def drain(t, slot):
        bb = t // seq
        s = t - bb * seq
        pltpu.make_async_copy(tblp_hbm.at[pidxs[slot]], pairs[slot],
                              sems[slot]).wait()
        @plsc.parallel_loop(0, 8, unroll=2)
        def _(g):
            hvec = hbufs[slot][pl.ds(g * _L, _L)]
            for tt in range(16):
                r = g * _L + tt
                h = hvec[tt]
                cvec = jnp.full((_L,), tt, jnp.int32) + (g * _L)
                for k in range(4):
                    vals = pairs[slot][r, pl.ds(h + 16 * k, _L)]
                    plsc.store_scatter(touts[slot],
                                       [iotav + (16 * k), cvec], vals)
        boff = pl.multiple_of((4 * wid + bb) * 128, 128)
        pltpu.sync_copy(touts[slot].at[:, pl.ds(0, 128)],
                        outt_hbm.at[s, :, pl.ds(boff, 128)])

    for b in range(2):
        stage_fire(b, b)

    def outer(kk, _):
        for b in range(2):
            t = 2 * kk + b
            drain(t, b)
            nt = t + 2

            @pl.when(nt < n_units)
            def _():
                stage_fire(nt, b)
        return 0

    lax.fori_loop(0, n_units // 2, outer, 0)


def kernel(input_ids, table):
    b, s = input_ids.shape
    v, d = table.shape
    n = b * s
    assert d == 64 and v % 2 == 0 and b % (128 * _NW) == 0

    tblt = table.T
    vt = (v // 128) * 128
    tail = jnp.pad(table[vt:].T, ((0, 0), (0, 128 - (v - vt))))
    ids = input_ids.reshape(n)

    mesh = plsc.VectorSubcoreMesh(core_axis_name="c", subcore_axis_name="s",
                                  num_cores=_NC, num_subcores=_NS)
    params = pltpu.CompilerParams(use_tc_tiling_on_sc=True, needs_layout_passes=False)

    k1 = pl.kernel(
        functools.partial(_k1_restructure, v),
        out_type=jax.ShapeDtypeStruct((v // 2, 128), jnp.float32),
        mesh=mesh,
        scratch_types=[
            [pltpu.VMEM((d, 137), jnp.float32)] * 2,
            [pltpu.VMEM((64, 128), jnp.float32)] * 2,
            [pltpu.SemaphoreType.DMA] * 2,
        ],
        compiler_params=params,
    )
    tblp = k1(tblt, tail)

    k2 = pl.kernel(
        functools.partial(_k2_gather, b, s, v),
        out_type=jax.ShapeDtypeStruct((s, d, b), jnp.float32),
        mesh=mesh,
        scratch_types=[
            pltpu.VMEM((n // _NW,), jnp.int32),
            [pltpu.VMEM((128,), jnp.int32)] * 2,
            [pltpu.VMEM((128,), jnp.int32)] * 2,
            [pltpu.VMEM((128, 128), jnp.float32)] * 2,
            [pltpu.VMEM((d, 137), jnp.float32)] * 2,
            [pltpu.SemaphoreType.DMA] * 2,
        ],
        compiler_params=params,
    )
    outt = k2(ids, tblp)
    return outt.transpose(2, 0, 1)


# speedup vs baseline: 1.6615x; 1.0711x over previous
import functools

import jax
import jax.numpy as jnp
from jax import lax
from jax.experimental import pallas as pl
from jax.experimental.pallas import tpu as pltpu
from jax.experimental.pallas import tpu_sc as plsc

_NC = 2
_NS = 16
_NW = _NC * _NS
_L = 16


def _k1_restructure(v, tblt_hbm, tail_hbm, tblp_hbm, tins, touts, sems):
    wid = lax.axis_index("s") * _NC + lax.axis_index("c")
    nu = v // 128
    t_max = nu // _NW + 2

    iotav = lax.iota(jnp.int32, _L)

    def fire(t, b):
        u = wid + _NW * t

        @pl.when(u < nu)
        def _():
            off = pl.multiple_of(128 * u, 128)
            pltpu.async_copy(tblt_hbm.at[:, pl.ds(off, 128)],
                             tins[b].at[:, pl.ds(0, 128)], sems[b])

    def compute(b, npp):
        @plsc.parallel_loop(0, npp, unroll=4)
        def _(pp):
            c0 = jnp.full((_L,), 2 * pp, jnp.int32)
            c1 = c0 + 1
            for k in range(4):
                rvec = iotav + (16 * k)
                touts[b][pp, pl.ds(16 * k, _L)] = plsc.load_gather(
                    tins[b], [rvec, c0])
                touts[b][pp, pl.ds(64 + 16 * k, _L)] = plsc.load_gather(
                    tins[b], [rvec, c1])

    for b in range(2):
        fire(b, b)

    def outer(kk, _):
        for b in range(2):
            t = 2 * kk + b
            u = wid + _NW * t

            @pl.when(u < nu)
            def _():
                pltpu.make_async_copy(
                    tblt_hbm.at[:, pl.ds(pl.multiple_of(128 * u, 128), 128)],
                    tins[b].at[:, pl.ds(0, 128)], sems[b]).wait()
                compute(b, 64)
                pltpu.sync_copy(
                    touts[b],
                    tblp_hbm.at[pl.ds(pl.multiple_of(64 * u, 8), 64)])

            fire(t + 2, b)
        return 0

    lax.fori_loop(0, t_max // 2, outer, 0)

    ntail = v - 128 * nu

    @pl.when(wid == 4)
    def _():
        pltpu.sync_copy(tail_hbm, tins[0].at[:, pl.ds(0, 128)])
        compute(0, ntail // 2)
        pltpu.sync_copy(
            touts[0].at[pl.ds(0, ntail // 2)],
            tblp_hbm.at[pl.ds(64 * nu, ntail // 2)])


def _k2_gather(batch, seq, vocab, ids_hbm, tblp_hbm, outt_hbm,
               idb, pidxs, hbufs, pairs, touts, sems):
    wid = lax.axis_index("s") * _NC + lax.axis_index("c")
    bpw = batch // _NW
    n_units = 4 * seq

    iotav = lax.iota(jnp.int32, _L)
    iota_s = iotav * seq

    pltpu.sync_copy(
        ids_hbm.at[pl.ds(pl.multiple_of(bpw * seq * wid, 1024), bpw * seq)],
        idb)

    def stage_fire(t, slot):
        bb = t // seq
        s = t - bb * seq
        for g in range(8):
            offs = iota_s + ((bb * 128 + 16 * g) * seq + s)
            idvec = plsc.load_gather(idb, [offs])
            idvec = jnp.where(idvec >= vocab, 0, idvec)
            pidxs[slot][pl.ds(16 * g, _L)] = idvec >> 1
            hbufs[slot][pl.ds(16 * g, _L)] = (idvec & 1) * 64
        pltpu.async_copy(tblp_hbm.at[pidxs[slot]], pairs[slot], sems[slot])

    def drain(t, slot):
        bb = t // seq
        s = t - bb * seq
        pltpu.make_async_copy(tblp_hbm.at[pidxs[slot]], pairs[slot],
                              sems[slot]).wait()
        hvecs = tuple(hbufs[slot][pl.ds(16 * g, _L)] for g in range(8))
        rvecs = tuple(iotav + (16 * g) for g in range(8))

        @plsc.parallel_loop(0, 64, unroll=8, carry=hvecs)
        def _(e, hv):
            for g in range(8):
                touts[slot][e, pl.ds(16 * g, _L)] = plsc.load_gather(
                    pairs[slot], [rvecs[g], hv[g] + e])
            return hv
        boff = pl.multiple_of((4 * wid + bb) * 128, 128)
        pltpu.sync_copy(touts[slot], outt_hbm.at[s, :, pl.ds(boff, 128)])

    for b in range(2):
        stage_fire(b, b)

    def outer(kk, _):
        for b in range(2):
            t = 2 * kk + b
            drain(t, b)
            nt = t + 2

            @pl.when(nt < n_units)
            def _():
                stage_fire(nt, b)
        return 0

    lax.fori_loop(0, n_units // 2, outer, 0)


def kernel(input_ids, table):
    b, s = input_ids.shape
    v, d = table.shape
    n = b * s
    assert d == 64 and v % 2 == 0 and b % (128 * _NW) == 0

    tblt = table.T
    vt = (v // 128) * 128
    tail = jnp.pad(table[vt:].T, ((0, 0), (0, 128 - (v - vt))))
    ids = input_ids.reshape(n)

    mesh = plsc.VectorSubcoreMesh(core_axis_name="c", subcore_axis_name="s",
                                  num_cores=_NC, num_subcores=_NS)
    params = pltpu.CompilerParams(use_tc_tiling_on_sc=True, needs_layout_passes=False)

    k1 = pl.kernel(
        functools.partial(_k1_restructure, v),
        out_type=jax.ShapeDtypeStruct((v // 2, 128), jnp.float32),
        mesh=mesh,
        scratch_types=[
            [pltpu.VMEM((d, 137), jnp.float32)] * 2,
            [pltpu.VMEM((64, 128), jnp.float32)] * 2,
            [pltpu.SemaphoreType.DMA] * 2,
        ],
        compiler_params=params,
    )
    tblp = k1(tblt, tail)

    k2 = pl.kernel(
        functools.partial(_k2_gather, b, s, v),
        out_type=jax.ShapeDtypeStruct((s, d, b), jnp.float32),
        mesh=mesh,
        scratch_types=[
            pltpu.VMEM((n // _NW,), jnp.int32),
            [pltpu.VMEM((128,), jnp.int32)] * 2,
            [pltpu.VMEM((128,), jnp.int32)] * 2,
            [pltpu.VMEM((128, 128), jnp.float32)] * 2,
            [pltpu.VMEM((d, 128), jnp.float32)] * 2,
            [pltpu.SemaphoreType.DMA] * 2,
        ],
        compiler_params=params,
    )
    outt = k2(ids, tblp)
    return outt.transpose(2, 0, 1)


# speedup vs baseline: 1.6631x; 1.0009x over previous
import functools

import jax
import jax.numpy as jnp
from jax import lax
from jax.experimental import pallas as pl
from jax.experimental.pallas import tpu as pltpu
from jax.experimental.pallas import tpu_sc as plsc

_NC = 2
_NS = 16
_NW = _NC * _NS
_L = 16


def _k1_restructure(v, tblt_hbm, tail_hbm, tblp_hbm, tins, touts, sems):
    wid = lax.axis_index("s") * _NC + lax.axis_index("c")
    nu = v // 128
    t_max = nu // _NW + 2

    iotav = lax.iota(jnp.int32, _L)

    def fire(t, b):
        u = wid + _NW * t

        @pl.when(u < nu)
        def _():
            off = pl.multiple_of(128 * u, 128)
            pltpu.async_copy(tblt_hbm.at[:, pl.ds(off, 128)],
                             tins[b].at[:, pl.ds(0, 128)], sems[b])

    def compute(b, npp):
        @plsc.parallel_loop(0, npp, unroll=8)
        def _(pp):
            c0 = jnp.full((_L,), 2 * pp, jnp.int32)
            c1 = c0 + 1
            for k in range(4):
                rvec = iotav + (16 * k)
                touts[b][pp, pl.ds(16 * k, _L)] = plsc.load_gather(
                    tins[b], [rvec, c0])
                touts[b][pp, pl.ds(64 + 16 * k, _L)] = plsc.load_gather(
                    tins[b], [rvec, c1])

    for b in range(2):
        fire(b, b)

    def outer(kk, _):
        for b in range(2):
            t = 2 * kk + b
            u = wid + _NW * t

            @pl.when(u < nu)
            def _():
                pltpu.make_async_copy(
                    tblt_hbm.at[:, pl.ds(pl.multiple_of(128 * u, 128), 128)],
                    tins[b].at[:, pl.ds(0, 128)], sems[b]).wait()
                compute(b, 64)
                pltpu.sync_copy(
                    touts[b],
                    tblp_hbm.at[pl.ds(pl.multiple_of(64 * u, 8), 64)])

            fire(t + 2, b)
        return 0

    lax.fori_loop(0, t_max // 2, outer, 0)

    ntail = v - 128 * nu

    @pl.when(wid == 4)
    def _():
        pltpu.sync_copy(tail_hbm, tins[0].at[:, pl.ds(0, 128)])
        compute(0, ntail // 2)
        pltpu.sync_copy(
            touts[0].at[pl.ds(0, ntail // 2)],
            tblp_hbm.at[pl.ds(64 * nu, ntail // 2)])


def _k2_gather(batch, seq, vocab, ids_hbm, tblp_hbm, outt_hbm,
               idb, pidxs, hbufs, pairs, touts, sems):
    wid = lax.axis_index("s") * _NC + lax.axis_index("c")
    bpw = batch // _NW
    n_units = 4 * seq

    iotav = lax.iota(jnp.int32, _L)
    iota_s = iotav * seq

    pltpu.sync_copy(
        ids_hbm.at[pl.ds(pl.multiple_of(bpw * seq * wid, 1024), bpw * seq)],
        idb)

    def stage_fire(t, slot):
        bb = t // seq
        s = t - bb * seq
        for g in range(8):
            offs = iota_s + ((bb * 128 + 16 * g) * seq + s)
            idvec = plsc.load_gather(idb, [offs])
            idvec = jnp.where(idvec >= vocab, 0, idvec)
            pidxs[slot][pl.ds(16 * g, _L)] = idvec >> 1
            hbufs[slot][pl.ds(16 * g, _L)] = (idvec & 1) * 64
        pltpu.async_copy(tblp_hbm.at[pidxs[slot]], pairs[slot], sems[slot])

    def drain(t, slot):
        bb = t // seq
        s = t - bb * seq
        pltpu.make_async_copy(tblp_hbm.at[pidxs[slot]], pairs[slot],
                              sems[slot]).wait()
        hvecs = tuple(hbufs[slot][pl.ds(16 * g, _L)] for g in range(8))
        rvecs = tuple(iotav + (16 * g) for g in range(8))

        @plsc.parallel_loop(0, 64, unroll=8, carry=hvecs)
        def _(e, hv):
            for g in range(8):
                touts[slot][e, pl.ds(16 * g, _L)] = plsc.load_gather(
                    pairs[slot], [rvecs[g], hv[g] + e])
            return hv
        boff = pl.multiple_of((4 * wid + bb) * 128, 128)
        pltpu.sync_copy(touts[slot], outt_hbm.at[s, :, pl.ds(boff, 128)])

    for b in range(2):
        stage_fire(b, b)

    def outer(kk, _):
        for b in range(2):
            t = 2 * kk + b
            drain(t, b)
            nt = t + 2

            @pl.when(nt < n_units)
            def _():
                stage_fire(nt, b)
        return 0

    lax.fori_loop(0, n_units // 2, outer, 0)


def kernel(input_ids, table):
    b, s = input_ids.shape
    v, d = table.shape
    n = b * s
    assert d == 64 and v % 2 == 0 and b % (128 * _NW) == 0

    tblt = table.T
    vt = (v // 128) * 128
    tail = jnp.pad(table[vt:].T, ((0, 0), (0, 128 - (v - vt))))
    ids = input_ids.reshape(n)

    mesh = plsc.VectorSubcoreMesh(core_axis_name="c", subcore_axis_name="s",
                                  num_cores=_NC, num_subcores=_NS)
    params = pltpu.CompilerParams(use_tc_tiling_on_sc=True, needs_layout_passes=False)

    k1 = pl.kernel(
        functools.partial(_k1_restructure, v),
        out_type=jax.ShapeDtypeStruct((v // 2, 128), jnp.float32),
        mesh=mesh,
        scratch_types=[
            [pltpu.VMEM((d, 137), jnp.float32)] * 2,
            [pltpu.VMEM((64, 128), jnp.float32)] * 2,
            [pltpu.SemaphoreType.DMA] * 2,
        ],
        compiler_params=params,
    )
    tblp = k1(tblt, tail)

    k2 = pl.kernel(
        functools.partial(_k2_gather, b, s, v),
        out_type=jax.ShapeDtypeStruct((s, d, b), jnp.float32),
        mesh=mesh,
        scratch_types=[
            pltpu.VMEM((n // _NW,), jnp.int32),
            [pltpu.VMEM((128,), jnp.int32)] * 2,
            [pltpu.VMEM((128,), jnp.int32)] * 2,
            [pltpu.VMEM((128, 128), jnp.float32)] * 2,
            [pltpu.VMEM((d, 128), jnp.float32)] * 2,
            [pltpu.SemaphoreType.DMA] * 2,
        ],
        compiler_params=params,
    )
    outt = k2(ids, tblp)
    return outt.transpose(2, 0, 1)


# speedup vs baseline: 2.1241x; 1.2772x over previous
import functools

import jax
import jax.numpy as jnp
from jax import lax
from jax.experimental import pallas as pl
from jax.experimental.pallas import tpu as pltpu
from jax.experimental.pallas import tpu_sc as plsc

_NC = 2
_NS = 16
_NW = _NC * _NS
_L = 16
_NBUF = 2


def _emb_kernel(n_total, vocab, d, chunk, ids_hbm, table_hbm, out_hbm,
                idx_v, rows_v, gsems):
    wid = lax.axis_index("s") * _NC + lax.axis_index("c")
    per_w = n_total // _NW
    base = wid * per_w
    n_ch = per_w // chunk

    def stage_and_fire(j, b):
        off = base + j * chunk
        pltpu.sync_copy(ids_hbm.at[pl.ds(off, chunk)], idx_v.at[b])

        def clamp_body(i, _):
            v = idx_v[b, pl.ds(i * _L, _L)]
            idx_v[b, pl.ds(i * _L, _L)] = jnp.where(v >= vocab, 0, v)
            return 0

        lax.fori_loop(0, chunk // _L, clamp_body, 0, unroll=4)
        pltpu.async_copy(table_hbm.at[idx_v.at[b]], rows_v.at[b], gsems[b])

    for b in range(_NBUF):
        stage_and_fire(b, b)

    def outer(k, _):
        j0 = k * _NBUF
        for b in range(_NBUF):
            j = j0 + b
            pltpu.make_async_copy(table_hbm.at[idx_v.at[b]], rows_v.at[b],
                                  gsems[b]).wait()
            pltpu.sync_copy(rows_v.at[b], out_hbm.at[pl.ds(base + j * chunk,
                                                           chunk)])
            nj = j + _NBUF

            @pl.when(nj < n_ch)
            def _():
                stage_and_fire(nj, b)

        return 0

    lax.fori_loop(0, n_ch // _NBUF, outer, 0)


def kernel(input_ids, table):
    b, s = input_ids.shape
    v, d = table.shape
    n = b * s
    ids = input_ids.reshape(n)

    chunk = 800
    assert n % (_NW * chunk * _NBUF) == 0

    mesh = plsc.VectorSubcoreMesh(core_axis_name="c", subcore_axis_name="s",
                                  num_cores=_NC, num_subcores=_NS)
    run = pl.kernel(
        functools.partial(_emb_kernel, n, v, d, chunk),
        out_type=jax.ShapeDtypeStruct((n, d), jnp.float32),
        mesh=mesh,
        scratch_types=[
            pltpu.VMEM((_NBUF, chunk), jnp.int32),
            pltpu.VMEM((_NBUF, chunk, d), jnp.float32),
            [pltpu.SemaphoreType.DMA] * _NBUF,
        ],
        compiler_params=pltpu.CompilerParams(use_tc_tiling_on_sc=False),
    )
    out = run(ids, table)
    return out.reshape(b, s, d)
